# Initial kernel scaffold; baseline (speedup 1.0000x reference)
import jax
import jax.numpy as jnp
from jax.experimental import pallas as pl


def kernel(x, edge_index, edge_attr, Wd, W1, W2, W3, W4, W5, W6):
    raise NotImplementedError("write your pallas kernel here")


# speedup vs baseline: 1.1643x; 1.1643x over previous
import functools

import jax
import jax.numpy as jnp
from jax import lax
from jax.experimental import pallas as pl
from jax.experimental.pallas import tpu as pltpu
from jax.experimental.pallas import tpu_sc as plsc

N = 100000
E = 1600000
L = 16
NC = 2
NS = 16
BATCH = 128
NB = E // BATCH
NB_PER_TILE = -(-NB // NS)
RCH = 1000
NCH = N // RCH
NCH_PER_TILE = -(-NCH // NS)


def _make_sc_agg(ncg):
  ncg_per_core = -(-ncg // NC)
  mesh = plsc.VectorSubcoreMesh(
      core_axis_name="c", subcore_axis_name="s",
      num_cores=NC, num_subcores=NS)

  @functools.partial(
      pl.kernel,
      out_type=jax.ShapeDtypeStruct((N, ncg, L), jnp.float32),
      mesh=mesh,
      scratch_types=[
          pltpu.MemorySpace.VMEM_SHARED((N, L), jnp.float32),
          pltpu.VMEM((BATCH,), jnp.int32),
          pltpu.VMEM((BATCH,), jnp.int32),
          pltpu.VMEM((BATCH,), jnp.int32),
          pltpu.VMEM((BATCH, L), jnp.float32),
          pltpu.SemaphoreType.DMA,
      ],
      compiler_params=pltpu.CompilerParams(use_tc_tiling_on_sc=False),
  )
  def sc_agg(x2, x3, src_hbm, dst_hbm, out_hbm,
             acc, src_v, gidx_v, didx_v, rows_v, sem):
    c = lax.axis_index("c")
    s = lax.axis_index("s")

    def cg_pass(i, carry):
      cg = c * ncg_per_core + i

      @pl.when(cg < ncg)
      def _():
        def init_chunk(j, carry2):
          ch = j * NS + s

          @pl.when(ch < NCH)
          def _():
            pltpu.sync_copy(x3.at[pl.ds(ch * RCH, RCH), cg],
                            acc.at[pl.ds(ch * RCH, RCH)])
          return carry2
        lax.fori_loop(0, NCH_PER_TILE, init_chunk, 0)
        plsc.subcore_barrier()

        def edge_step(j, carry2):
          b = j * NS + s

          @pl.when(b < NB)
          def _():
            e0 = b * BATCH
            pltpu.sync_copy(src_hbm.at[pl.ds(e0, BATCH)], src_v)
            pltpu.sync_copy(dst_hbm.at[pl.ds(e0, BATCH)], didx_v)
            for l in range(BATCH // L):
              sl = pl.ds(l * L, L)
              gidx_v[sl] = src_v[sl] * ncg + cg
            pltpu.async_copy(x2.at[gidx_v], rows_v, sem).wait()
            pltpu.sync_copy(rows_v, acc.at[didx_v], add=True)
          return carry2
        lax.fori_loop(0, NB_PER_TILE, edge_step, 0)
        plsc.subcore_barrier()

        def flush_chunk(j, carry2):
          ch = j * NS + s

          @pl.when(ch < NCH)
          def _():
            pltpu.sync_copy(acc.at[pl.ds(ch * RCH, RCH)],
                            out_hbm.at[pl.ds(ch * RCH, RCH), cg])
          return carry2
        lax.fori_loop(0, NCH_PER_TILE, flush_chunk, 0)
        plsc.subcore_barrier()
      return carry

    lax.fori_loop(0, ncg_per_core, cg_pass, 0)

  return sc_agg


_sc_agg3 = _make_sc_agg(3)
_sc_agg8 = _make_sc_agg(8)


def _agg(x, src, dst, ncg):
  sc = _sc_agg3 if ncg == 3 else _sc_agg8
  x2 = x.reshape(N * ncg, L)
  x3 = x.reshape(N, ncg, L)
  out = sc(x2, x3, src, dst)
  return out.reshape(N, ncg * L)


R_TC = 2000


def _mm1_body(a_ref, w1_ref, x_ref, wd_ref, o_ref):
  y = jnp.dot(a_ref[...], w1_ref[...], preferred_element_type=jnp.float32)
  z = jnp.dot(x_ref[...], wd_ref[...], preferred_element_type=jnp.float32)
  o_ref[...] = jnp.maximum(y, 0.0) + z


def _mm1(agg, W1, x, Wd):
  return pl.pallas_call(
      _mm1_body,
      grid=(N // R_TC,),
      in_specs=[
          pl.BlockSpec((R_TC, 48), lambda i: (i, 0)),
          pl.BlockSpec((48, 128), lambda i: (0, 0)),
          pl.BlockSpec((R_TC, 48), lambda i: (i, 0)),
          pl.BlockSpec((48, 128), lambda i: (0, 0)),
      ],
      out_specs=pl.BlockSpec((R_TC, 128), lambda i: (i, 0)),
      out_shape=jax.ShapeDtypeStruct((N, 128), jnp.float32),
  )(agg, W1, x, Wd)


def _mm_body(a_ref, w_ref, r_ref, o_ref):
  y = jnp.dot(a_ref[...], w_ref[...], preferred_element_type=jnp.float32)
  o_ref[...] = jnp.maximum(y, 0.0) + r_ref[...]


def _mm(agg, W, res):
  return pl.pallas_call(
      _mm_body,
      grid=(N // R_TC,),
      in_specs=[
          pl.BlockSpec((R_TC, 128), lambda i: (i, 0)),
          pl.BlockSpec((128, 128), lambda i: (0, 0)),
          pl.BlockSpec((R_TC, 128), lambda i: (i, 0)),
      ],
      out_specs=pl.BlockSpec((R_TC, 128), lambda i: (i, 0)),
      out_shape=jax.ShapeDtypeStruct((N, 128), jnp.float32),
  )(agg, W, res)


def kernel(x, edge_index, edge_attr, Wd, W1, W2, W3, W4, W5, W6):
  src = edge_index[0].astype(jnp.int32)
  dst = edge_index[1].astype(jnp.int32)
  x1 = _mm1(_agg(x, src, dst, 3), W1, x, Wd)
  x2 = _mm(_agg(x1, src, dst, 8), W2, x1)
  x3 = _mm(_agg(x2, src, dst, 8), W3, x2)
  x4 = _mm(_agg(x3, src, dst, 8), W4, x3)
  x5 = _mm(_agg(x4, src, dst, 8), W5, x4)
  x6 = _mm(_agg(x5, src, dst, 8), W6, x5)
  return x6


# speedup vs baseline: 3.7533x; 3.2236x over previous
import functools

import jax
import jax.numpy as jnp
from jax import lax
from jax.experimental import pallas as pl
from jax.experimental.pallas import tpu as pltpu
from jax.experimental.pallas import tpu_sc as plsc

N = 100000
E = 1600000
L = 16
NC = 2
NS = 16
BATCH = 128
NB = E // BATCH
NB_PER_TILE = -(-NB // NS)
DB = 4
NG = -(-NB_PER_TILE // DB)
RCH = 1000
NCH = N // RCH
NCH_PER_TILE = -(-NCH // NS)


def _make_sc_agg(ncg):
  ncg_per_core = -(-ncg // NC)
  mesh = plsc.VectorSubcoreMesh(
      core_axis_name="c", subcore_axis_name="s",
      num_cores=NC, num_subcores=NS)

  @functools.partial(
      pl.kernel,
      out_type=jax.ShapeDtypeStruct((N, ncg, L), jnp.float32),
      mesh=mesh,
      scratch_types=[
          pltpu.MemorySpace.VMEM_SHARED((N, L), jnp.float32),
          pltpu.VMEM((2, DB, BATCH), jnp.int32),
          pltpu.VMEM((DB, BATCH), jnp.int32),
          pltpu.VMEM((2, DB, BATCH), jnp.int32),
          pltpu.VMEM((DB, BATCH, L), jnp.float32),
      ]
      + [pltpu.SemaphoreType.DMA] * (3 * DB),
      compiler_params=pltpu.CompilerParams(use_tc_tiling_on_sc=False),
  )
  def sc_agg(x2, x3, src_hbm, dst_hbm, out_hbm,
             acc, src_v, gidx_v, didx_v, rows_v, *sems):
    sem_i = sems[0:DB]
    sem_g = sems[DB:2 * DB]
    sem_s = sems[2 * DB:3 * DB]
    c = lax.axis_index("c")
    s = lax.axis_index("s")

    def issue_idx(j, p, t):
      b = j * NS + s

      @pl.when(b < NB)
      def _():
        e0 = b * BATCH
        pltpu.async_copy(src_hbm.at[pl.ds(e0, BATCH)], src_v.at[p, t],
                         sem_i[t])
        pltpu.async_copy(dst_hbm.at[pl.ds(e0, BATCH)], didx_v.at[p, t],
                         sem_i[t])

    def cg_pass(i, carry):
      cg = c * ncg_per_core + i

      @pl.when(cg < ncg)
      def _():
        def init_chunk(j, carry2):
          ch = j * NS + s

          @pl.when(ch < NCH)
          def _():
            pltpu.sync_copy(x3.at[pl.ds(ch * RCH, RCH), cg],
                            acc.at[pl.ds(ch * RCH, RCH)])
          return carry2
        lax.fori_loop(0, NCH_PER_TILE, init_chunk, 0)
        plsc.subcore_barrier()

        for t in range(DB):
          issue_idx(t, 0, t)

        def one_group(g, p):
          j0 = g * DB
          for t in range(DB):
            b_prev = (j0 - DB + t) * NS + s

            @pl.when((j0 + t >= DB) & (b_prev < NB))
            def _(t=t, b_prev=b_prev):
              pltpu.make_async_copy(
                  rows_v.at[t], acc.at[didx_v.at[1 - p, t]], sem_s[t]).wait()
          gathers = []
          for t in range(DB):
            j = j0 + t
            b = j * NS + s
            cond = b < NB

            @pl.when(cond)
            def _(t=t, b=b):
              e0 = b * BATCH
              pltpu.make_async_copy(
                  src_hbm.at[pl.ds(e0, BATCH)], src_v.at[p, t],
                  sem_i[t]).wait()
              pltpu.make_async_copy(
                  dst_hbm.at[pl.ds(e0, BATCH)], didx_v.at[p, t],
                  sem_i[t]).wait()
              for l in range(BATCH // L):
                sl = pl.ds(l * L, L)
                gidx_v[t, sl] = src_v[p, t, sl] * ncg + cg
              pltpu.async_copy(x2.at[gidx_v.at[t]], rows_v.at[t], sem_g[t])
            gathers.append(cond)
          for t in range(DB):
            issue_idx(j0 + DB + t, 1 - p, t)
          for t in range(DB):
            @pl.when(gathers[t])
            def _(t=t):
              pltpu.make_async_copy(
                  x2.at[gidx_v.at[t]], rows_v.at[t], sem_g[t]).wait()
              pltpu.async_copy(rows_v.at[t], acc.at[didx_v.at[p, t]],
                               sem_s[t], add=True)

        def group_pair(gg, carry2):
          one_group(gg * 2, 0)
          one_group(gg * 2 + 1, 1)
          return carry2
        lax.fori_loop(0, -(-NG // 2), group_pair, 0)
        n_groups = -(-NG // 2) * 2
        for t in range(DB):
          b_last = ((n_groups - 1) * DB + t) * NS + s

          @pl.when(b_last < NB)
          def _(t=t, b_last=b_last):
            pltpu.make_async_copy(
                rows_v.at[t], acc.at[didx_v.at[(n_groups - 1) % 2, t]],
                sem_s[t]).wait()
        plsc.subcore_barrier()

        def flush_chunk(j, carry2):
          ch = j * NS + s

          @pl.when(ch < NCH)
          def _():
            pltpu.sync_copy(acc.at[pl.ds(ch * RCH, RCH)],
                            out_hbm.at[pl.ds(ch * RCH, RCH), cg])
          return carry2
        lax.fori_loop(0, NCH_PER_TILE, flush_chunk, 0)
        plsc.subcore_barrier()
      return carry

    lax.fori_loop(0, ncg_per_core, cg_pass, 0)

  return sc_agg


_sc_agg3 = _make_sc_agg(3)
_sc_agg8 = _make_sc_agg(8)


def _agg(x, src, dst, ncg):
  sc = _sc_agg3 if ncg == 3 else _sc_agg8
  x2 = x.reshape(N * ncg, L)
  x3 = x.reshape(N, ncg, L)
  out = sc(x2, x3, src, dst)
  return out.reshape(N, ncg * L)


R_TC = 2000


def _mm1_body(a_ref, w1_ref, x_ref, wd_ref, o_ref):
  y = jnp.dot(a_ref[...], w1_ref[...], preferred_element_type=jnp.float32)
  z = jnp.dot(x_ref[...], wd_ref[...], preferred_element_type=jnp.float32)
  o_ref[...] = jnp.maximum(y, 0.0) + z


def _mm1(agg, W1, x, Wd):
  return pl.pallas_call(
      _mm1_body,
      grid=(N // R_TC,),
      in_specs=[
          pl.BlockSpec((R_TC, 48), lambda i: (i, 0)),
          pl.BlockSpec((48, 128), lambda i: (0, 0)),
          pl.BlockSpec((R_TC, 48), lambda i: (i, 0)),
          pl.BlockSpec((48, 128), lambda i: (0, 0)),
      ],
      out_specs=pl.BlockSpec((R_TC, 128), lambda i: (i, 0)),
      out_shape=jax.ShapeDtypeStruct((N, 128), jnp.float32),
  )(agg, W1, x, Wd)


def _mm_body(a_ref, w_ref, r_ref, o_ref):
  y = jnp.dot(a_ref[...], w_ref[...], preferred_element_type=jnp.float32)
  o_ref[...] = jnp.maximum(y, 0.0) + r_ref[...]


def _mm(agg, W, res):
  return pl.pallas_call(
      _mm_body,
      grid=(N // R_TC,),
      in_specs=[
          pl.BlockSpec((R_TC, 128), lambda i: (i, 0)),
          pl.BlockSpec((128, 128), lambda i: (0, 0)),
          pl.BlockSpec((R_TC, 128), lambda i: (i, 0)),
      ],
      out_specs=pl.BlockSpec((R_TC, 128), lambda i: (i, 0)),
      out_shape=jax.ShapeDtypeStruct((N, 128), jnp.float32),
  )(agg, W, res)


def kernel(x, edge_index, edge_attr, Wd, W1, W2, W3, W4, W5, W6):
  src = edge_index[0].astype(jnp.int32)
  dst = edge_index[1].astype(jnp.int32)
  x1 = _mm1(_agg(x, src, dst, 3), W1, x, Wd)
  x2 = _mm(_agg(x1, src, dst, 8), W2, x1)
  x3 = _mm(_agg(x2, src, dst, 8), W3, x2)
  x4 = _mm(_agg(x3, src, dst, 8), W4, x3)
  x5 = _mm(_agg(x4, src, dst, 8), W5, x4)
  x6 = _mm(_agg(x5, src, dst, 8), W6, x5)
  return x6


# speedup vs baseline: 4.1671x; 1.1102x over previous
import functools

import jax
import jax.numpy as jnp
from jax import lax
from jax.experimental import pallas as pl
from jax.experimental.pallas import tpu as pltpu
from jax.experimental.pallas import tpu_sc as plsc

N = 100000
E = 1600000
L = 16
NC = 2
NS = 16
BATCH = 128
NB = E // BATCH
NB_PER_TILE = -(-NB // NS)
DB = 8
NG = -(-NB_PER_TILE // DB)
RCH = 1000
NCH = N // RCH
NCH_PER_TILE = -(-NCH // NS)


def _make_sc_agg(ncg):
  ncg_per_core = -(-ncg // NC)
  mesh = plsc.VectorSubcoreMesh(
      core_axis_name="c", subcore_axis_name="s",
      num_cores=NC, num_subcores=NS)

  @functools.partial(
      pl.kernel,
      out_type=jax.ShapeDtypeStruct((N, ncg, L), jnp.float32),
      mesh=mesh,
      scratch_types=[
          pltpu.MemorySpace.VMEM_SHARED((N, L), jnp.float32),
          pltpu.VMEM((2, DB, BATCH), jnp.int32),
          pltpu.VMEM((DB, BATCH), jnp.int32),
          pltpu.VMEM((2, DB, BATCH), jnp.int32),
          pltpu.VMEM((DB, BATCH, L), jnp.float32),
      ]
      + [pltpu.SemaphoreType.DMA] * (3 * DB),
      compiler_params=pltpu.CompilerParams(use_tc_tiling_on_sc=False),
  )
  def sc_agg(x2, x3, src_hbm, dst_hbm, out_hbm,
             acc, src_v, gidx_v, didx_v, rows_v, *sems):
    sem_i = sems[0:DB]
    sem_g = sems[DB:2 * DB]
    sem_s = sems[2 * DB:3 * DB]
    c = lax.axis_index("c")
    s = lax.axis_index("s")

    def issue_idx(j, p, t):
      b = j * NS + s

      @pl.when(b < NB)
      def _():
        e0 = b * BATCH
        pltpu.async_copy(src_hbm.at[pl.ds(e0, BATCH)], src_v.at[p, t],
                         sem_i[t])
        pltpu.async_copy(dst_hbm.at[pl.ds(e0, BATCH)], didx_v.at[p, t],
                         sem_i[t])

    def cg_pass(i, carry):
      cg = c * ncg_per_core + i
      col0 = cg * L

      @pl.when(cg < ncg)
      def _():
        def init_chunk(j, carry2):
          ch = j * NS + s

          @pl.when(ch < NCH)
          def _():
            pltpu.sync_copy(x3.at[pl.ds(ch * RCH, RCH), cg],
                            acc.at[pl.ds(ch * RCH, RCH)])
          return carry2
        lax.fori_loop(0, NCH_PER_TILE, init_chunk, 0)
        plsc.subcore_barrier()

        for t in range(DB):
          issue_idx(t, 0, t)

        def one_group(g, p):
          j0 = g * DB
          for t in range(DB):
            b_prev = (j0 - DB + t) * NS + s

            @pl.when((j0 + t >= DB) & (b_prev < NB))
            def _(t=t, b_prev=b_prev):
              pltpu.make_async_copy(
                  rows_v.at[t], acc.at[didx_v.at[1 - p, t]], sem_s[t]).wait()
          gathers = []
          for t in range(DB):
            j = j0 + t
            b = j * NS + s
            cond = b < NB

            @pl.when(cond)
            def _(t=t, b=b):
              e0 = b * BATCH
              pltpu.make_async_copy(
                  src_hbm.at[pl.ds(e0, BATCH)], src_v.at[p, t],
                  sem_i[t]).wait()
              pltpu.make_async_copy(
                  dst_hbm.at[pl.ds(e0, BATCH)], didx_v.at[p, t],
                  sem_i[t]).wait()
              for l in range(BATCH // L):
                sl = pl.ds(l * L, L)
                gidx_v[t, sl] = src_v[p, t, sl] * ncg + cg
              pltpu.async_copy(x2.at[gidx_v.at[t]], rows_v.at[t], sem_g[t])
            gathers.append(cond)
          for t in range(DB):
            issue_idx(j0 + DB + t, 1 - p, t)
          for t in range(DB):
            @pl.when(gathers[t])
            def _(t=t):
              pltpu.make_async_copy(
                  x2.at[gidx_v.at[t]], rows_v.at[t], sem_g[t]).wait()
              pltpu.async_copy(rows_v.at[t], acc.at[didx_v.at[p, t]],
                               sem_s[t], add=True)

        def group_pair(gg, carry2):
          one_group(gg * 2, 0)
          one_group(gg * 2 + 1, 1)
          return carry2
        lax.fori_loop(0, -(-NG // 2), group_pair, 0)
        n_groups = -(-NG // 2) * 2
        for t in range(DB):
          b_last = ((n_groups - 1) * DB + t) * NS + s

          @pl.when(b_last < NB)
          def _(t=t, b_last=b_last):
            pltpu.make_async_copy(
                rows_v.at[t], acc.at[didx_v.at[(n_groups - 1) % 2, t]],
                sem_s[t]).wait()
        plsc.subcore_barrier()

        def flush_chunk(j, carry2):
          ch = j * NS + s

          @pl.when(ch < NCH)
          def _():
            pltpu.sync_copy(acc.at[pl.ds(ch * RCH, RCH)],
                            out_hbm.at[pl.ds(ch * RCH, RCH), cg])
          return carry2
        lax.fori_loop(0, NCH_PER_TILE, flush_chunk, 0)
        plsc.subcore_barrier()
      return carry

    lax.fori_loop(0, ncg_per_core, cg_pass, 0)

  return sc_agg


_sc_agg3 = _make_sc_agg(3)
_sc_agg8 = _make_sc_agg(8)


def _agg(x, src, dst, ncg):
  sc = _sc_agg3 if ncg == 3 else _sc_agg8
  out = sc(x.reshape(N * ncg, L), x.reshape(N, ncg, L), src, dst)
  return out.reshape(N, ncg * L)


R_TC = 2000


def _mm1_body(a_ref, w1_ref, x_ref, wd_ref, o_ref):
  y = jnp.dot(a_ref[...], w1_ref[...], preferred_element_type=jnp.float32)
  z = jnp.dot(x_ref[...], wd_ref[...], preferred_element_type=jnp.float32)
  o_ref[...] = jnp.maximum(y, 0.0) + z


def _mm1(agg, W1, x, Wd):
  return pl.pallas_call(
      _mm1_body,
      grid=(N // R_TC,),
      in_specs=[
          pl.BlockSpec((R_TC, 48), lambda i: (i, 0)),
          pl.BlockSpec((48, 128), lambda i: (0, 0)),
          pl.BlockSpec((R_TC, 48), lambda i: (i, 0)),
          pl.BlockSpec((48, 128), lambda i: (0, 0)),
      ],
      out_specs=pl.BlockSpec((R_TC, 128), lambda i: (i, 0)),
      out_shape=jax.ShapeDtypeStruct((N, 128), jnp.float32),
  )(agg, W1, x, Wd)


def _mm_body(a_ref, w_ref, r_ref, o_ref):
  y = jnp.dot(a_ref[...], w_ref[...], preferred_element_type=jnp.float32)
  o_ref[...] = jnp.maximum(y, 0.0) + r_ref[...]


def _mm(agg, W, res):
  return pl.pallas_call(
      _mm_body,
      grid=(N // R_TC,),
      in_specs=[
          pl.BlockSpec((R_TC, 128), lambda i: (i, 0)),
          pl.BlockSpec((128, 128), lambda i: (0, 0)),
          pl.BlockSpec((R_TC, 128), lambda i: (i, 0)),
      ],
      out_specs=pl.BlockSpec((R_TC, 128), lambda i: (i, 0)),
      out_shape=jax.ShapeDtypeStruct((N, 128), jnp.float32),
  )(agg, W, res)


def kernel(x, edge_index, edge_attr, Wd, W1, W2, W3, W4, W5, W6):
  src = edge_index[0].astype(jnp.int32)
  dst = edge_index[1].astype(jnp.int32)
  x1 = _mm1(_agg(x, src, dst, 3), W1, x, Wd)
  x2 = _mm(_agg(x1, src, dst, 8), W2, x1)
  x3 = _mm(_agg(x2, src, dst, 8), W3, x2)
  x4 = _mm(_agg(x3, src, dst, 8), W4, x3)
  x5 = _mm(_agg(x4, src, dst, 8), W5, x4)
  x6 = _mm(_agg(x5, src, dst, 8), W6, x5)
  return x6


# speedup vs baseline: 4.3337x; 1.0400x over previous
import functools

import jax
import jax.numpy as jnp
from jax import lax
from jax.experimental import pallas as pl
from jax.experimental.pallas import tpu as pltpu
from jax.experimental.pallas import tpu_sc as plsc

N = 100000
E = 1600000
L = 16
NC = 2
NS = 16
BATCH = 128
DB = 8
NB = -(-E // (BATCH * NS * DB)) * NS * DB
E_PAD = NB * BATCH
NB_PER_TILE = NB // NS
NG = NB_PER_TILE // DB
NPAD_ROWS = 8
RCH = 1000
NCH = N // RCH
NCH_PER_TILE = -(-NCH // NS)


def _make_sc_agg(ncg):
  ncg_per_core = -(-ncg // NC)
  mesh = plsc.VectorSubcoreMesh(
      core_axis_name="c", subcore_axis_name="s",
      num_cores=NC, num_subcores=NS)

  @functools.partial(
      pl.kernel,
      out_type=jax.ShapeDtypeStruct((N, ncg, L), jnp.float32),
      mesh=mesh,
      scratch_types=[
          pltpu.MemorySpace.VMEM_SHARED((N + NPAD_ROWS, L), jnp.float32),
          pltpu.VMEM((2, DB, BATCH), jnp.int32),
          pltpu.VMEM((DB, BATCH), jnp.int32),
          pltpu.VMEM((2, DB, BATCH), jnp.int32),
          pltpu.VMEM((DB, BATCH, L), jnp.float32),
      ]
      + [pltpu.SemaphoreType.DMA] * (1 + 2 * DB),
      compiler_params=pltpu.CompilerParams(use_tc_tiling_on_sc=False),
  )
  def sc_agg(x2, x3, src_hbm, dst_hbm, out_hbm,
             acc, src_v, gidx_v, didx_v, rows_v, *sems):
    sem_i = sems[0]
    sem_g = sems[1:1 + DB]
    sem_s = sems[1 + DB:1 + 2 * DB]
    c = lax.axis_index("c")
    s = lax.axis_index("s")

    def issue_idx(g, p):
      b0 = s * NB_PER_TILE + g * DB
      pltpu.async_copy(src_hbm.at[pl.ds(b0, DB)], src_v.at[p], sem_i)
      pltpu.async_copy(dst_hbm.at[pl.ds(b0, DB)], didx_v.at[p], sem_i)

    def cg_pass(i, carry):
      cg = c * ncg_per_core + i
      col0 = cg * L

      @pl.when(cg < ncg)
      def _():
        def init_chunk(j, carry2):
          ch = j * NS + s

          @pl.when(ch < NCH)
          def _():
            pltpu.sync_copy(x3.at[pl.ds(ch * RCH, RCH), cg],
                            acc.at[pl.ds(ch * RCH, RCH)])
          return carry2
        lax.fori_loop(0, NCH_PER_TILE, init_chunk, 0)
        plsc.subcore_barrier()

        issue_idx(0, 0)

        def one_group(g, p, first, last):
          if not first:
            for t in range(DB):
              pltpu.make_async_copy(
                  rows_v.at[t], acc.at[didx_v.at[1 - p, t]], sem_s[t]).wait()
          pltpu.make_async_copy(
              src_hbm.at[pl.ds(0, DB)], src_v.at[p], sem_i).wait()
          pltpu.make_async_copy(
              dst_hbm.at[pl.ds(0, DB)], didx_v.at[p], sem_i).wait()
          for t in range(DB):
            for l in range(BATCH // L):
              sl = pl.ds(l * L, L)
              gidx_v[t, sl] = src_v[p, t, sl] * ncg + cg
            pltpu.async_copy(x2.at[gidx_v.at[t]], rows_v.at[t], sem_g[t])
          if not last:
            issue_idx(g + 1, 1 - p)
          for t in range(DB):
            pltpu.make_async_copy(
                x2.at[gidx_v.at[t]], rows_v.at[t], sem_g[t]).wait()
            pltpu.async_copy(rows_v.at[t], acc.at[didx_v.at[p, t]],
                             sem_s[t], add=True)

        one_group(0, 0, True, False)

        def group_pair(gg, carry2):
          g = gg * 2 + 1
          one_group(g, 1, False, False)
          one_group(g + 1, 0, False, False)
          return carry2
        lax.fori_loop(0, (NG - 1) // 2, group_pair, 0)
        one_group(NG - 1, 1, False, True)
        for t in range(DB):
          pltpu.make_async_copy(
              rows_v.at[t], acc.at[didx_v.at[1, t]], sem_s[t]).wait()
        plsc.subcore_barrier()

        def flush_chunk(j, carry2):
          ch = j * NS + s

          @pl.when(ch < NCH)
          def _():
            pltpu.sync_copy(acc.at[pl.ds(ch * RCH, RCH)],
                            out_hbm.at[pl.ds(ch * RCH, RCH), cg])
          return carry2
        lax.fori_loop(0, NCH_PER_TILE, flush_chunk, 0)
        plsc.subcore_barrier()
      return carry

    lax.fori_loop(0, ncg_per_core, cg_pass, 0)

  return sc_agg


_sc_agg3 = _make_sc_agg(3)
_sc_agg8 = _make_sc_agg(8)


def _agg(x, src2d, dst2d, ncg):
  sc = _sc_agg3 if ncg == 3 else _sc_agg8
  out = sc(x.reshape(N * ncg, L), x.reshape(N, ncg, L), src2d, dst2d)
  return out.reshape(N, ncg * L)


R_TC = 2000


def _mm1_body(a_ref, w1_ref, x_ref, wd_ref, o_ref):
  y = jnp.dot(a_ref[...], w1_ref[...], preferred_element_type=jnp.float32)
  z = jnp.dot(x_ref[...], wd_ref[...], preferred_element_type=jnp.float32)
  o_ref[...] = jnp.maximum(y, 0.0) + z


def _mm1(agg, W1, x, Wd):
  return pl.pallas_call(
      _mm1_body,
      grid=(N // R_TC,),
      in_specs=[
          pl.BlockSpec((R_TC, 48), lambda i: (i, 0)),
          pl.BlockSpec((48, 128), lambda i: (0, 0)),
          pl.BlockSpec((R_TC, 48), lambda i: (i, 0)),
          pl.BlockSpec((48, 128), lambda i: (0, 0)),
      ],
      out_specs=pl.BlockSpec((R_TC, 128), lambda i: (i, 0)),
      out_shape=jax.ShapeDtypeStruct((N, 128), jnp.float32),
  )(agg, W1, x, Wd)


def _mm_body(a_ref, w_ref, r_ref, o_ref):
  y = jnp.dot(a_ref[...], w_ref[...], preferred_element_type=jnp.float32)
  o_ref[...] = jnp.maximum(y, 0.0) + r_ref[...]


def _mm(agg, W, res):
  return pl.pallas_call(
      _mm_body,
      grid=(N // R_TC,),
      in_specs=[
          pl.BlockSpec((R_TC, 128), lambda i: (i, 0)),
          pl.BlockSpec((128, 128), lambda i: (0, 0)),
          pl.BlockSpec((R_TC, 128), lambda i: (i, 0)),
      ],
      out_specs=pl.BlockSpec((R_TC, 128), lambda i: (i, 0)),
      out_shape=jax.ShapeDtypeStruct((N, 128), jnp.float32),
  )(agg, W, res)


def kernel(x, edge_index, edge_attr, Wd, W1, W2, W3, W4, W5, W6):
  npad = E_PAD - E
  pad_iota = jnp.arange(npad, dtype=jnp.int32)
  src = jnp.concatenate(
      [edge_index[0].astype(jnp.int32), (pad_iota * 769) % N])
  dst = jnp.concatenate(
      [edge_index[1].astype(jnp.int32), N + (pad_iota % NPAD_ROWS)])
  src = src.reshape(NB, BATCH)
  dst = dst.reshape(NB, BATCH)
  x1 = _mm1(_agg(x, src, dst, 3), W1, x, Wd)
  x2 = _mm(_agg(x1, src, dst, 8), W2, x1)
  x3 = _mm(_agg(x2, src, dst, 8), W3, x2)
  x4 = _mm(_agg(x3, src, dst, 8), W4, x3)
  x5 = _mm(_agg(x4, src, dst, 8), W5, x4)
  x6 = _mm(_agg(x5, src, dst, 8), W6, x5)
  return x6
